# baseline (speedup 1.0000x reference)
import functools

import jax
import jax.numpy as jnp
from jax import lax
from jax.experimental import pallas as pl
from jax.experimental.pallas import tpu as pltpu
from jax.experimental.pallas import tpu_sc as plsc

F = 26
VOCAB = 100000
EMB = 16
BATCH = 16384
D1 = 32
D2 = 32

ROWS = BATCH * F
NC, NS = 2, 16
NW = NC * NS
RPW = ROWS // NW
CH = 3328
NCH = RPW // CH


def _gather_body(sec_hbm, first_hbm, idx_hbm, sec_out, first_out,
                 idx_v, rows_v, frows_v, sem, fsem):
    wid = lax.axis_index("s") * NC + lax.axis_index("c")
    base = wid * RPW
    for c in range(NCH):
        off = base + c * CH
        pltpu.sync_copy(idx_hbm.at[pl.ds(off, CH)], idx_v)
        g1 = pltpu.async_copy(sec_hbm.at[idx_v], rows_v, sem)
        g2 = pltpu.async_copy(first_hbm.at[idx_v], frows_v, fsem)
        g1.wait()
        g2.wait()
        pltpu.sync_copy(rows_v, sec_out.at[pl.ds(off, CH)])
        pltpu.sync_copy(frows_v, first_out.at[pl.ds(off, CH)])


_gather = pl.kernel(
    _gather_body,
    mesh=plsc.VectorSubcoreMesh(core_axis_name="c", subcore_axis_name="s"),
    compiler_params=pltpu.CompilerParams(use_tc_tiling_on_sc=False),
    out_type=(
        jax.ShapeDtypeStruct((ROWS, EMB), jnp.float32),
        jax.ShapeDtypeStruct((ROWS,), jnp.float32),
    ),
    scratch_types=[
        pltpu.VMEM((CH,), jnp.int32),
        pltpu.VMEM((CH, EMB), jnp.float32),
        pltpu.VMEM((CH,), jnp.float32),
        pltpu.SemaphoreType.DMA,
        pltpu.SemaphoreType.DMA,
    ],
)


BS = 1024


def _mlp_body(first_ref, xv_ref, sec_ref, s_ref, w1a_ref, w1b_ref, b1_ref,
              w2_ref, b2_ref, w3_ref, b3_ref, out_ref):
    xv = xv_ref[:, :]
    fo = first_ref[:, :] * xv
    xvrep = jnp.dot(xv, s_ref[:, :], preferred_element_type=jnp.float32)
    so = sec_ref[:, :] * xvrep
    h = (jnp.dot(fo, w1a_ref[:, :], preferred_element_type=jnp.float32)
         + jnp.dot(so, w1b_ref[:, :], preferred_element_type=jnp.float32)
         + b1_ref[:, :])
    h = jnp.maximum(h, 0.0)
    h = jnp.maximum(
        jnp.dot(h, w2_ref[:, :], preferred_element_type=jnp.float32)
        + b2_ref[:, :], 0.0)
    out_ref[:, :] = (jnp.dot(h, w3_ref[:, :], preferred_element_type=jnp.float32)
                     + b3_ref[:, :])


def _mlp(first_g, xv, sec_g, s_mat, w1a, w1b, b1e, W2, b2, W3, b3):
    grid = (BATCH // BS,)
    zero2 = lambda i: (0, 0)
    return pl.pallas_call(
        _mlp_body,
        grid=grid,
        in_specs=[
            pl.BlockSpec((BS, F), lambda i: (i, 0)),
            pl.BlockSpec((BS, F), lambda i: (i, 0)),
            pl.BlockSpec((BS, F * EMB), lambda i: (i, 0)),
            pl.BlockSpec((F, F * EMB), zero2),
            pl.BlockSpec((F, D1), zero2),
            pl.BlockSpec((F * EMB, D1), zero2),
            pl.BlockSpec((1, D1), zero2),
            pl.BlockSpec((D1, D2), zero2),
            pl.BlockSpec((1, D2), zero2),
            pl.BlockSpec((D2, 1), zero2),
            pl.BlockSpec((1, 1), zero2),
        ],
        out_specs=pl.BlockSpec((BS, 1), lambda i: (i, 0)),
        out_shape=jax.ShapeDtypeStruct((BATCH, 1), jnp.float32),
    )(first_g, xv, sec_g, s_mat, w1a, w1b, b1e, W2, b2, W3, b3)


def kernel(Xi, Xv, fm_bias, first_tables, second_tables, W1, b1, W2, b2, W3, b3):
    idx = (Xi[:, :, 0].astype(jnp.int32)
           + (jnp.arange(F, dtype=jnp.int32) * VOCAB)[None, :]).reshape(ROWS)
    sec_flat = second_tables.reshape(F * VOCAB, EMB)
    first_flat = first_tables.reshape(F * VOCAB)

    sec_g, first_g = _gather(sec_flat, first_flat, idx)

    s_mat = jnp.repeat(jnp.eye(F, dtype=jnp.float32), EMB, axis=1)
    w1a = W1[1:1 + F, :]
    w1b = W1[1 + F:, :]
    b1e = (b1 + fm_bias * W1[0, :]).reshape(1, D1)
    out = _mlp(first_g.reshape(BATCH, F), Xv.astype(jnp.float32),
               sec_g.reshape(BATCH, F * EMB), s_mat, w1a, w1b, b1e,
               W2, b2.reshape(1, D2), W3, b3.reshape(1, 1))
    return out.reshape(BATCH)
